# baseline (speedup 1.0000x reference)
import functools

import jax
import jax.numpy as jnp
from jax import lax
from jax.experimental import pallas as pl
from jax.experimental.pallas import tpu as pltpu
from jax.experimental.pallas import tpu_sc as plsc

N = 10000
E = 160000
NT = 16
PT = E // NT
BLK = 128
NBLK = 80
NPAD = 10032
RPT = NPAD // NT
SLICE = 640
CW = 128
F32 = jnp.float32

_K = 3
_M = 6


def _slice_start(sid):
    st = sid * RPT
    st = (st // 8) * 8
    return jnp.minimum(st, NPAD - SLICE)


def _sc_mesh():
    return plsc.VectorSubcoreMesh(core_axis_name="c", subcore_axis_name="s")


def _deg_body(slab, ones_h, zeros_h, out, idx_v, ones_v, acc, sem):
    core = lax.axis_index("c")
    sid = lax.axis_index("s")
    pltpu.sync_copy(slab.at[core, sid], idx_v)
    pltpu.sync_copy(ones_h, ones_v)
    st = _slice_start(sid)
    pltpu.sync_copy(zeros_h, acc.at[pl.ds(st, SLICE)])
    plsc.subcore_barrier()

    def start(j):
        pltpu.async_copy(ones_v, acc.at[idx_v.at[j]], sem, add=True)

    def drain(j):
        pltpu.make_async_copy(ones_v, acc.at[idx_v.at[j]], sem).wait()

    W = 8
    for j in range(W):
        start(j)

    def step(g, carry):
        j = W + g
        drain(j - W)
        start(j)
        return carry

    lax.fori_loop(0, NBLK - W, step, 0)
    for j in range(NBLK - W, NBLK):
        drain(j)
    plsc.subcore_barrier()
    pltpu.sync_copy(acc.at[pl.ds(st, SLICE)],
                    out.at[core, pl.ds(st, SLICE)])


def _deg_call(slab, ones_h, zeros_h):
    return pl.kernel(
        _deg_body,
        out_type=jax.ShapeDtypeStruct((2, NPAD, CW), F32),
        mesh=_sc_mesh(),
        scratch_types=[
            pltpu.VMEM((NBLK, BLK), jnp.int32),
            pltpu.VMEM((BLK, CW), F32),
            pltpu.VMEM_SHARED((NPAD, CW), F32),
            pltpu.SemaphoreType.DMA,
        ],
    )(slab, ones_h, zeros_h)


def _mp_body(P, tab, pair_slab, zeros_h, out,
             r0, r1, r2, q0, q1, q2, q3, q4, q5, acc, gsems, ssems, psems):
    rows = [r0, r1, r2]
    pairs = [q0, q1, q2, q3, q4, q5]
    core = lax.axis_index("c")
    sid = lax.axis_index("s")

    for p in range(P):
        chunk = 2 * p + core
        base = pair_slab.at[p, core, sid]

        def p_start(j, m):
            pltpu.async_copy(base.at[j], pairs[m], psems.at[m])

        def p_wait(j, m):
            pltpu.make_async_copy(base.at[j], pairs[m], psems.at[m]).wait()

        def g_start(b, m):
            pltpu.async_copy(tab.at[pairs[m].at[0]], rows[b], gsems.at[b])

        def g_wait(b, m):
            pltpu.make_async_copy(tab.at[pairs[m].at[0]], rows[b],
                                  gsems.at[b]).wait()

        def s_start(b, m):
            pltpu.async_copy(rows[b], acc.at[pairs[m].at[1]], ssems.at[b],
                             add=True)

        def s_wait(b, m):
            pltpu.make_async_copy(rows[b], acc.at[pairs[m].at[1]],
                                  ssems.at[b]).wait()

        st = _slice_start(sid)
        if p:
            plsc.subcore_barrier()
        pltpu.sync_copy(zeros_h, acc.at[pl.ds(st, SLICE)])
        plsc.subcore_barrier()

        for j in range(3):
            p_start(j, j)
        p_wait(0, 0); g_start(0, 0); p_start(3, 3)
        p_wait(1, 1); g_start(1, 1); p_start(4, 4)
        g_wait(0, 0); s_start(0, 0)
        p_wait(2, 2); g_start(2, 2); p_start(5, 5)
        g_wait(1, 1); s_start(1, 1)

        def step6(g, carry):
            j0 = 3 + 6 * g
            for u in range(6):
                ju = j0 + u
                b, m = u % _K, (3 + u) % _M
                s_wait(u % _K, u % _M)
                p_start(ju + 3, u % _M)
                p_wait(ju, m)
                g_start(b, m)
                g_wait((u + 2) % _K, (2 + u) % _M)
                s_start((u + 2) % _K, (2 + u) % _M)
            return carry

        lax.fori_loop(0, 12, step6, 0)

        for j in range(75, NBLK):
            s_wait((j - 3) % _K, (j - 3) % _M)
            if j + 3 < NBLK:
                p_start(j + 3, (j + 3) % _M)
            p_wait(j, j % _M)
            g_start(j % _K, j % _M)
            g_wait((j - 1) % _K, (j - 1) % _M)
            s_start((j - 1) % _K, (j - 1) % _M)
        g_wait((NBLK - 1) % _K, (NBLK - 1) % _M)
        s_start((NBLK - 1) % _K, (NBLK - 1) % _M)
        for j in range(NBLK - 3, NBLK):
            s_wait(j % _K, j % _M)

        plsc.subcore_barrier()
        pltpu.sync_copy(acc.at[pl.ds(st, SLICE)],
                        out.at[chunk, pl.ds(st, SLICE)])


def _mp_call(P, tab, pair_slab, zeros_h):
    nch = 2 * P
    return pl.kernel(
        functools.partial(_mp_body, P),
        out_type=jax.ShapeDtypeStruct((nch, NPAD, CW), F32),
        mesh=_sc_mesh(),
        scratch_types=(
            [pltpu.VMEM((BLK, CW), F32)] * _K
            + [pltpu.VMEM((2, BLK), jnp.int32)] * _M
            + [
                pltpu.VMEM_SHARED((NPAD, CW), F32),
                pltpu.SemaphoreType.DMA((_K,)),
                pltpu.SemaphoreType.DMA((_K,)),
                pltpu.SemaphoreType.DMA((_M,)),
            ]
        ),
    )(tab, pair_slab, zeros_h)


def _prescale_body(x_ref, deg_ref, o_ref):
    s = lax.rsqrt(jnp.maximum(deg_ref[...], 1.0))
    o_ref[0] = x_ref[...] * s


def _prescale(x, deg_out, n_chunks, nb, bn):
    return pl.pallas_call(
        _prescale_body,
        grid=(nb, n_chunks),
        in_specs=[
            pl.BlockSpec((bn, CW), lambda i, c: (i, c)),
            pl.BlockSpec((bn, 1), lambda i, c: (i, 0)),
        ],
        out_specs=pl.BlockSpec((1, bn, CW), lambda i, c: (c, i, 0)),
        out_shape=jax.ShapeDtypeStruct((n_chunks, N, CW), F32),
    )(x, deg_out)


def _dense_body(nch_in, out_chunks, agg_ref, w_ref, b_ref, g_ref, be_ref,
                a_ref, din_ref, dout_ref, o_ref):
    t = jnp.concatenate([agg_ref[c] for c in range(nch_in)], axis=1)
    acc = jnp.dot(t, w_ref[...], preferred_element_type=F32)
    s_in = lax.rsqrt(jnp.maximum(din_ref[...], 1.0))
    acc = acc * s_in + b_ref[...]
    mu = jnp.mean(acc, axis=-1, keepdims=True)
    d = acc - mu
    var = jnp.mean(d * d, axis=-1, keepdims=True)
    u = d * lax.rsqrt(var + 1e-5) * g_ref[...] + be_ref[...]
    r = jnp.where(u >= 0, u, a_ref[0, 0] * u)
    if out_chunks:
        s_out = lax.rsqrt(jnp.maximum(dout_ref[...], 1.0))
        r = r * s_out
        for c2 in range(out_chunks):
            o_ref[c2] = r[:, c2 * CW:(c2 + 1) * CW]
    else:
        o_ref[...] = r


def _dense(agg, W, b, g, be, a, deg_in, deg_out, out_chunks, nb, bn):
    nch_in, _, _ = agg.shape
    dout = W.shape[1]
    if out_chunks:
        out_shape = jax.ShapeDtypeStruct((out_chunks, N, CW), F32)
        out_spec = pl.BlockSpec((out_chunks, bn, CW), lambda i: (0, i, 0))
    else:
        out_shape = jax.ShapeDtypeStruct((N, dout), F32)
        out_spec = pl.BlockSpec((bn, dout), lambda i: (i, 0))
    return pl.pallas_call(
        functools.partial(_dense_body, nch_in, out_chunks),
        grid=(nb,),
        in_specs=[
            pl.BlockSpec((nch_in, bn, CW), lambda i: (0, i, 0)),
            pl.BlockSpec(W.shape, lambda i: (0, 0)),
            pl.BlockSpec((1, dout), lambda i: (0, 0)),
            pl.BlockSpec((1, dout), lambda i: (0, 0)),
            pl.BlockSpec((1, dout), lambda i: (0, 0)),
            pl.BlockSpec((1, 1), lambda i: (0, 0)),
            pl.BlockSpec((bn, 1), lambda i: (i, 0)),
            pl.BlockSpec((bn, 1), lambda i: (i, 0)),
        ],
        out_specs=out_spec,
        out_shape=out_shape,
    )(agg, W, b, g, be, a, deg_in, deg_out)


def _tile_slabs(idx, pad_vals):
    tiles = idx.reshape(NT, PT)
    pad = jnp.broadcast_to(pad_vals[None, :], (NT, pad_vals.shape[0]))
    return jnp.concatenate([tiles, pad], axis=1).reshape(NT, NBLK, BLK)


def kernel(x, edge_index, W1, b1, g1, be1, a1, W2, b2, g2, be2, a2):
    src = edge_index[0].astype(jnp.int32)
    dst = edge_index[1].astype(jnp.int32)
    npad = NBLK * BLK - PT

    trash = N + (jnp.arange(npad, dtype=jnp.int32) % (NPAD - N))
    spread = (jnp.arange(npad, dtype=jnp.int32) * 37) % N
    src_t = _tile_slabs(src, trash)
    src_g = _tile_slabs(src, spread)
    dst_t = _tile_slabs(dst, trash)

    deg_slab = jnp.stack([src_t, dst_t])
    ones_cw = jnp.ones((BLK, CW), F32)
    zeros_cw = jnp.zeros((SLICE, CW), F32)

    degs = _deg_call(deg_slab, ones_cw, zeros_cw)
    deg_out = degs[0, :N, 0:1]
    deg_in = degs[1, :N, 0:1]

    def pair_slab(P):
        offs = (jnp.arange(2 * P, dtype=jnp.int32).reshape(P, 2) * N)
        so = src_g[None, None] + offs[:, :, None, None, None]
        db = jnp.broadcast_to(dst_t[None, None], so.shape)
        return jnp.stack([so, db], axis=4)

    nb, bn = 25, 400

    hs = _prescale(x, deg_out, 2, nb, bn)
    agg1 = _mp_call(1, hs.reshape(2 * N, CW), pair_slab(1), zeros_cw)
    agg1 = agg1[:, :N, :]

    h1s = _dense(agg1, W1, b1.reshape(1, -1), g1.reshape(1, -1),
                 be1.reshape(1, -1), a1.reshape(1, 1), deg_in, deg_out,
                 out_chunks=4, nb=nb, bn=bn)

    agg2 = _mp_call(2, h1s.reshape(4 * N, CW), pair_slab(2), zeros_cw)
    agg2 = agg2[:, :N, :]

    out = _dense(agg2, W2, b2.reshape(1, -1), g2.reshape(1, -1),
                 be2.reshape(1, -1), a2.reshape(1, 1), deg_in, deg_in,
                 out_chunks=0, nb=nb, bn=bn)
    return out


# speedup vs baseline: 1.0956x; 1.0956x over previous
import functools

import numpy as _np

import jax
import jax.numpy as jnp
from jax import lax
from jax.experimental import pallas as pl
from jax.experimental.pallas import tpu as pltpu
from jax.experimental.pallas import tpu_sc as plsc

N = 10000
E = 160000
NT = 16
PT = E // NT
BLK = 128
NBLK = 80
NPAD = 10032
RPT = NPAD // NT
SLICE = 640
CW = 128
F32 = jnp.float32

_K = 3
_M = 6


def _slice_start(sid):
    st = sid * RPT
    st = (st // 8) * 8
    return jnp.minimum(st, NPAD - SLICE)


def _sc_mesh():
    return plsc.VectorSubcoreMesh(core_axis_name="c", subcore_axis_name="s")


def _deg_body(slab, ones_h, zeros_h, out, idx_v, ones_v, acc, sem):
    core = lax.axis_index("c")
    sid = lax.axis_index("s")
    pltpu.sync_copy(slab.at[sid], idx_v)
    pltpu.sync_copy(ones_h, ones_v)
    st = _slice_start(sid)
    pltpu.sync_copy(zeros_h, acc.at[pl.ds(st, SLICE)])
    plsc.subcore_barrier()

    def start(j):
        pltpu.async_copy(ones_v, acc.at[idx_v.at[j, core]], sem, add=True)

    def drain(j):
        pltpu.make_async_copy(ones_v, acc.at[idx_v.at[j, core]], sem).wait()

    W = 8
    for j in range(W):
        start(j)

    def step(g, carry):
        j = W + g
        drain(j - W)
        start(j)
        return carry

    lax.fori_loop(0, NBLK - W, step, 0)
    for j in range(NBLK - W, NBLK):
        drain(j)
    plsc.subcore_barrier()
    pltpu.sync_copy(acc.at[pl.ds(st, SLICE)],
                    out.at[core, pl.ds(st, SLICE)])


def _deg_call(slab, ones_h, zeros_h):
    return pl.kernel(
        _deg_body,
        out_type=jax.ShapeDtypeStruct((2, NPAD, CW), F32),
        mesh=_sc_mesh(),
        scratch_types=[
            pltpu.VMEM((NBLK, 2, BLK), jnp.int32),
            pltpu.VMEM((BLK, CW), F32),
            pltpu.VMEM_SHARED((NPAD, CW), F32),
            pltpu.SemaphoreType.DMA,
        ],
    )(slab, ones_h, zeros_h)


def _mp_body(P, tab, pair_slab, zeros_h, out,
             r0, r1, r2, q0, q1, q2, q3, q4, q5, acc, gsems, ssems, psems):
    rows = [r0, r1, r2]
    pairs = [q0, q1, q2, q3, q4, q5]
    core = lax.axis_index("c")
    sid = lax.axis_index("s")

    base = pair_slab.at[sid]
    for p in range(P):
        chunk = 2 * p + core

        def p_start(j, m):
            pltpu.async_copy(base.at[j], pairs[m], psems.at[m])

        def p_wait(j, m):
            pltpu.make_async_copy(base.at[j], pairs[m], psems.at[m]).wait()

        def g_start(b, m):
            pltpu.async_copy(tab.at[chunk].at[pairs[m].at[0]], rows[b],
                             gsems.at[b])

        def g_wait(b, m):
            pltpu.make_async_copy(tab.at[chunk].at[pairs[m].at[0]], rows[b],
                                  gsems.at[b]).wait()

        def s_start(b, m):
            pltpu.async_copy(rows[b], acc.at[pairs[m].at[1]], ssems.at[b],
                             add=True)

        def s_wait(b, m):
            pltpu.make_async_copy(rows[b], acc.at[pairs[m].at[1]],
                                  ssems.at[b]).wait()

        st = _slice_start(sid)
        if p:
            plsc.subcore_barrier()
        pltpu.sync_copy(zeros_h, acc.at[pl.ds(st, SLICE)])
        plsc.subcore_barrier()

        for j in range(3):
            p_start(j, j)
        p_wait(0, 0); g_start(0, 0); p_start(3, 3)
        p_wait(1, 1); g_start(1, 1); p_start(4, 4)
        p_wait(2, 2); g_start(2, 2); p_start(5, 5)
        g_wait(0, 0); s_start(0, 0)

        def step6(g, carry):
            j0 = 3 + 6 * g
            for u in range(6):
                ju = j0 + u
                b, m = u % _K, (3 + u) % _M
                s_wait(u % _K, u % _M)
                p_start(ju + 3, u % _M)
                p_wait(ju, m)
                g_start(b, m)
                g_wait((u + 1) % _K, (1 + u) % _M)
                s_start((u + 1) % _K, (1 + u) % _M)
            return carry

        lax.fori_loop(0, 12, step6, 0)

        for j in range(75, NBLK):
            s_wait((j - 3) % _K, (j - 3) % _M)
            if j + 3 < NBLK:
                p_start(j + 3, (j + 3) % _M)
            p_wait(j, j % _M)
            g_start(j % _K, j % _M)
            g_wait((j - 2) % _K, (j - 2) % _M)
            s_start((j - 2) % _K, (j - 2) % _M)
        for j in range(NBLK - 2, NBLK):
            g_wait(j % _K, j % _M)
            s_start(j % _K, j % _M)
        for j in range(NBLK - 3, NBLK):
            s_wait(j % _K, j % _M)

        plsc.subcore_barrier()
        pltpu.sync_copy(acc.at[pl.ds(st, SLICE)],
                        out.at[chunk, pl.ds(st, SLICE)])


def _mp_call(P, tab, pair_slab, zeros_h):
    nch = 2 * P
    return pl.kernel(
        functools.partial(_mp_body, P),
        out_type=jax.ShapeDtypeStruct((nch, NPAD, CW), F32),
        mesh=_sc_mesh(),
        scratch_types=(
            [pltpu.VMEM((BLK, CW), F32)] * _K
            + [pltpu.VMEM((2, BLK), jnp.int32)] * _M
            + [
                pltpu.VMEM_SHARED((NPAD, CW), F32),
                pltpu.SemaphoreType.DMA((_K,)),
                pltpu.SemaphoreType.DMA((_K,)),
                pltpu.SemaphoreType.DMA((_M,)),
            ]
        ),
    )(tab, pair_slab, zeros_h)


def _prescale_body(x_ref, degs_ref, corr_ref, o_ref):
    d = degs_ref[0][:, 0:1] - corr_ref[...]
    s = lax.rsqrt(jnp.maximum(d, 1.0))
    o_ref[0] = x_ref[...] * s


def _prescale(x, degs, corr, n_chunks, nb, bn):
    return pl.pallas_call(
        _prescale_body,
        grid=(nb, n_chunks),
        in_specs=[
            pl.BlockSpec((bn, CW), lambda i, c: (i, c)),
            pl.BlockSpec((1, bn, CW), lambda i, c: (0, i, 0)),
            pl.BlockSpec((bn, 1), lambda i, c: (i, 0)),
        ],
        out_specs=pl.BlockSpec((1, bn, CW), lambda i, c: (c, i, 0)),
        out_shape=jax.ShapeDtypeStruct((n_chunks, N, CW), F32),
    )(x, degs, corr)


def _dense_body(nch_in, out_chunks, agg_ref, w_ref, b_ref, g_ref, be_ref,
                a_ref, degs_ref, corr_ref, o_ref):
    t = jnp.concatenate([agg_ref[c] for c in range(nch_in)], axis=1)
    acc = jnp.dot(t, w_ref[...], preferred_element_type=F32)
    s_in = lax.rsqrt(jnp.maximum(degs_ref[1][:, 0:1], 1.0))
    acc = acc * s_in + b_ref[...]
    mu = jnp.mean(acc, axis=-1, keepdims=True)
    d = acc - mu
    var = jnp.mean(d * d, axis=-1, keepdims=True)
    u = d * lax.rsqrt(var + 1e-5) * g_ref[...] + be_ref[...]
    r = jnp.where(u >= 0, u, a_ref[0, 0] * u)
    if out_chunks:
        s_out = lax.rsqrt(jnp.maximum(degs_ref[0][:, 0:1] - corr_ref[...], 1.0))
        r = r * s_out
        for c2 in range(out_chunks):
            o_ref[c2] = r[:, c2 * CW:(c2 + 1) * CW]
    else:
        o_ref[...] = r


def _dense(agg, W, b, g, be, a, degs, corr, out_chunks, nb, bn):
    nch_in = agg.shape[0]
    dout = W.shape[1]
    if out_chunks:
        out_shape = jax.ShapeDtypeStruct((out_chunks, N, CW), F32)
        out_spec = pl.BlockSpec((out_chunks, bn, CW), lambda i: (0, i, 0))
    else:
        out_shape = jax.ShapeDtypeStruct((N, dout), F32)
        out_spec = pl.BlockSpec((bn, dout), lambda i: (i, 0))
    return pl.pallas_call(
        functools.partial(_dense_body, nch_in, out_chunks),
        grid=(nb,),
        in_specs=[
            pl.BlockSpec((nch_in, bn, CW), lambda i: (0, i, 0)),
            pl.BlockSpec(W.shape, lambda i: (0, 0)),
            pl.BlockSpec((1, dout), lambda i: (0, 0)),
            pl.BlockSpec((1, dout), lambda i: (0, 0)),
            pl.BlockSpec((1, dout), lambda i: (0, 0)),
            pl.BlockSpec((1, 1), lambda i: (0, 0)),
            pl.BlockSpec((2, bn, CW), lambda i: (0, i, 0)),
            pl.BlockSpec((bn, 1), lambda i: (i, 0)),
        ],
        out_specs=out_spec,
        out_shape=out_shape,
    )(agg, W, b, g, be, a, degs, corr)


def _tile_slabs(idx, pad_vals):
    tiles = idx.reshape(NT, PT)
    pad = jnp.broadcast_to(pad_vals[None, :], (NT, pad_vals.shape[0]))
    return jnp.concatenate([tiles, pad], axis=1).reshape(NT, NBLK, BLK)


_NSLOT = NBLK * BLK - PT
_SPREAD = (_np.arange(_NSLOT) * 37) % N
_CORR = _np.zeros((N, 1), _np.float32)
_CORR[_SPREAD, 0] = float(NT)


def kernel(x, edge_index, W1, b1, g1, be1, a1, W2, b2, g2, be2, a2):
    src = edge_index[0].astype(jnp.int32)
    dst = edge_index[1].astype(jnp.int32)

    trash = N + (jnp.arange(_NSLOT, dtype=jnp.int32) % (NPAD - N))
    spread = jnp.asarray(_SPREAD, dtype=jnp.int32)
    src_g = _tile_slabs(src, spread)
    dst_t = _tile_slabs(dst, trash)
    pair = jnp.stack([src_g, dst_t], axis=2)

    ones_cw = jnp.ones((BLK, CW), F32)
    zeros_cw = jnp.zeros((SLICE, CW), F32)
    corr = jnp.asarray(_CORR)

    degs = _deg_call(pair, ones_cw, zeros_cw)
    nb, bn = 25, 400

    hs = _prescale(x, degs, corr, 2, nb, bn)
    agg1 = _mp_call(1, hs, pair, zeros_cw)

    h1s = _dense(agg1, W1, b1.reshape(1, -1), g1.reshape(1, -1),
                 be1.reshape(1, -1), a1.reshape(1, 1), degs, corr,
                 out_chunks=4, nb=nb, bn=bn)

    agg2 = _mp_call(2, h1s, pair, zeros_cw)

    out = _dense(agg2, W2, b2.reshape(1, -1), g2.reshape(1, -1),
                 be2.reshape(1, -1), a2.reshape(1, 1), degs, corr,
                 out_chunks=0, nb=nb, bn=bn)
    return out


# speedup vs baseline: 1.0962x; 1.0006x over previous
import functools

import numpy as _np

import jax
import jax.numpy as jnp
from jax import lax
from jax.experimental import pallas as pl
from jax.experimental.pallas import tpu as pltpu
from jax.experimental.pallas import tpu_sc as plsc

N = 10000
E = 160000
NT = 16
PT = E // NT
BLK = 128
NBLK = 80
NPAD = 10032
RPT = NPAD // NT
SLICE = 640
CW = 128
F32 = jnp.float32

_K = 3
_M = 6


def _slice_start(sid):
    st = sid * RPT
    st = (st // 8) * 8
    return jnp.minimum(st, NPAD - SLICE)


def _sc_mesh():
    return plsc.VectorSubcoreMesh(core_axis_name="c", subcore_axis_name="s")


def _deg_body(slab, ones_h, zeros_h, out, idx_v, ones_v, acc, sem):
    core = lax.axis_index("c")
    sid = lax.axis_index("s")
    pltpu.sync_copy(slab.at[sid], idx_v)
    pltpu.sync_copy(ones_h, ones_v)
    st = _slice_start(sid)
    pltpu.sync_copy(zeros_h, acc.at[pl.ds(st, SLICE)])
    plsc.subcore_barrier()

    def start(j):
        pltpu.async_copy(ones_v, acc.at[idx_v.at[j, core]], sem, add=True)

    def drain(j):
        pltpu.make_async_copy(ones_v, acc.at[idx_v.at[j, core]], sem).wait()

    W = 8
    for j in range(W):
        start(j)

    def step(g, carry):
        j = W + g
        drain(j - W)
        start(j)
        return carry

    lax.fori_loop(0, NBLK - W, step, 0)
    for j in range(NBLK - W, NBLK):
        drain(j)
    plsc.subcore_barrier()
    pltpu.sync_copy(acc.at[pl.ds(st, SLICE)],
                    out.at[core, pl.ds(st, SLICE)])


def _deg_call(slab, ones_h, zeros_h):
    return pl.kernel(
        _deg_body,
        out_type=jax.ShapeDtypeStruct((2, NPAD, CW), F32),
        mesh=_sc_mesh(),
        scratch_types=[
            pltpu.VMEM((NBLK, 2, BLK), jnp.int32),
            pltpu.VMEM((BLK, CW), F32),
            pltpu.VMEM_SHARED((NPAD, CW), F32),
            pltpu.SemaphoreType.DMA,
        ],
    )(slab, ones_h, zeros_h)


def _mp_body(P, tab, pair_slab, zeros_h, out,
             r0, r1, r2, q0, q1, q2, q3, q4, q5, acc, gsems, ssems, psems):
    rows = [r0, r1, r2]
    pairs = [q0, q1, q2, q3, q4, q5]
    core = lax.axis_index("c")
    sid = lax.axis_index("s")

    base = pair_slab.at[sid]
    for p in range(P):
        chunk = 2 * p + core

        def p_start(j, m):
            pltpu.async_copy(base.at[j], pairs[m], psems.at[m])

        def p_wait(j, m):
            pltpu.make_async_copy(base.at[j], pairs[m], psems.at[m]).wait()

        def g_start(b, m):
            for h in range(2):
                pltpu.async_copy(
                    tab.at[chunk].at[pairs[m].at[0, pl.ds(h * 64, 64)]],
                    rows[b].at[pl.ds(h * 64, 64)], gsems.at[b])

        def g_wait(b, m):
            for h in range(2):
                pltpu.make_async_copy(
                    tab.at[chunk].at[pairs[m].at[0, pl.ds(h * 64, 64)]],
                    rows[b].at[pl.ds(h * 64, 64)], gsems.at[b]).wait()

        def s_start(b, m):
            pltpu.async_copy(rows[b], acc.at[pairs[m].at[1]], ssems.at[b],
                             add=True)

        def s_wait(b, m):
            pltpu.make_async_copy(rows[b], acc.at[pairs[m].at[1]],
                                  ssems.at[b]).wait()

        st = _slice_start(sid)
        if p:
            plsc.subcore_barrier()
        pltpu.sync_copy(zeros_h, acc.at[pl.ds(st, SLICE)])
        plsc.subcore_barrier()

        for j in range(3):
            p_start(j, j)
        p_wait(0, 0); g_start(0, 0); p_start(3, 3)
        p_wait(1, 1); g_start(1, 1); p_start(4, 4)
        p_wait(2, 2); g_start(2, 2); p_start(5, 5)
        g_wait(0, 0); s_start(0, 0)

        def step6(g, carry):
            j0 = 3 + 6 * g
            for u in range(6):
                ju = j0 + u
                b, m = u % _K, (3 + u) % _M
                s_wait(u % _K, u % _M)
                p_start(ju + 3, u % _M)
                p_wait(ju, m)
                g_start(b, m)
                g_wait((u + 1) % _K, (1 + u) % _M)
                s_start((u + 1) % _K, (1 + u) % _M)
            return carry

        lax.fori_loop(0, 12, step6, 0)

        for j in range(75, NBLK):
            s_wait((j - 3) % _K, (j - 3) % _M)
            if j + 3 < NBLK:
                p_start(j + 3, (j + 3) % _M)
            p_wait(j, j % _M)
            g_start(j % _K, j % _M)
            g_wait((j - 2) % _K, (j - 2) % _M)
            s_start((j - 2) % _K, (j - 2) % _M)
        for j in range(NBLK - 2, NBLK):
            g_wait(j % _K, j % _M)
            s_start(j % _K, j % _M)
        for j in range(NBLK - 3, NBLK):
            s_wait(j % _K, j % _M)

        plsc.subcore_barrier()
        pltpu.sync_copy(acc.at[pl.ds(st, SLICE)],
                        out.at[chunk, pl.ds(st, SLICE)])


def _mp_call(P, tab, pair_slab, zeros_h):
    nch = 2 * P
    return pl.kernel(
        functools.partial(_mp_body, P),
        out_type=jax.ShapeDtypeStruct((nch, NPAD, CW), F32),
        mesh=_sc_mesh(),
        scratch_types=(
            [pltpu.VMEM((BLK, CW), F32)] * _K
            + [pltpu.VMEM((2, BLK), jnp.int32)] * _M
            + [
                pltpu.VMEM_SHARED((NPAD, CW), F32),
                pltpu.SemaphoreType.DMA((_K,)),
                pltpu.SemaphoreType.DMA((_K,)),
                pltpu.SemaphoreType.DMA((_M,)),
            ]
        ),
    )(tab, pair_slab, zeros_h)


def _prescale_body(x_ref, degs_ref, corr_ref, o_ref):
    d = degs_ref[0][:, 0:1] - corr_ref[...]
    s = lax.rsqrt(jnp.maximum(d, 1.0))
    o_ref[0] = x_ref[...] * s


def _prescale(x, degs, corr, n_chunks, nb, bn):
    return pl.pallas_call(
        _prescale_body,
        grid=(nb, n_chunks),
        in_specs=[
            pl.BlockSpec((bn, CW), lambda i, c: (i, c)),
            pl.BlockSpec((1, bn, CW), lambda i, c: (0, i, 0)),
            pl.BlockSpec((bn, 1), lambda i, c: (i, 0)),
        ],
        out_specs=pl.BlockSpec((1, bn, CW), lambda i, c: (c, i, 0)),
        out_shape=jax.ShapeDtypeStruct((n_chunks, N, CW), F32),
    )(x, degs, corr)


def _dense_body(nch_in, out_chunks, agg_ref, w_ref, b_ref, g_ref, be_ref,
                a_ref, degs_ref, corr_ref, o_ref):
    t = jnp.concatenate([agg_ref[c] for c in range(nch_in)], axis=1)
    acc = jnp.dot(t, w_ref[...], preferred_element_type=F32)
    s_in = lax.rsqrt(jnp.maximum(degs_ref[1][:, 0:1], 1.0))
    acc = acc * s_in + b_ref[...]
    mu = jnp.mean(acc, axis=-1, keepdims=True)
    d = acc - mu
    var = jnp.mean(d * d, axis=-1, keepdims=True)
    u = d * lax.rsqrt(var + 1e-5) * g_ref[...] + be_ref[...]
    r = jnp.where(u >= 0, u, a_ref[0, 0] * u)
    if out_chunks:
        s_out = lax.rsqrt(jnp.maximum(degs_ref[0][:, 0:1] - corr_ref[...], 1.0))
        r = r * s_out
        for c2 in range(out_chunks):
            o_ref[c2] = r[:, c2 * CW:(c2 + 1) * CW]
    else:
        o_ref[...] = r


def _dense(agg, W, b, g, be, a, degs, corr, out_chunks, nb, bn):
    nch_in = agg.shape[0]
    dout = W.shape[1]
    if out_chunks:
        out_shape = jax.ShapeDtypeStruct((out_chunks, N, CW), F32)
        out_spec = pl.BlockSpec((out_chunks, bn, CW), lambda i: (0, i, 0))
    else:
        out_shape = jax.ShapeDtypeStruct((N, dout), F32)
        out_spec = pl.BlockSpec((bn, dout), lambda i: (i, 0))
    return pl.pallas_call(
        functools.partial(_dense_body, nch_in, out_chunks),
        grid=(nb,),
        in_specs=[
            pl.BlockSpec((nch_in, bn, CW), lambda i: (0, i, 0)),
            pl.BlockSpec(W.shape, lambda i: (0, 0)),
            pl.BlockSpec((1, dout), lambda i: (0, 0)),
            pl.BlockSpec((1, dout), lambda i: (0, 0)),
            pl.BlockSpec((1, dout), lambda i: (0, 0)),
            pl.BlockSpec((1, 1), lambda i: (0, 0)),
            pl.BlockSpec((2, bn, CW), lambda i: (0, i, 0)),
            pl.BlockSpec((bn, 1), lambda i: (i, 0)),
        ],
        out_specs=out_spec,
        out_shape=out_shape,
    )(agg, W, b, g, be, a, degs, corr)


def _tile_slabs(idx, pad_vals):
    tiles = idx.reshape(NT, PT)
    pad = jnp.broadcast_to(pad_vals[None, :], (NT, pad_vals.shape[0]))
    return jnp.concatenate([tiles, pad], axis=1).reshape(NT, NBLK, BLK)


_NSLOT = NBLK * BLK - PT
_SPREAD = (_np.arange(_NSLOT) * 37) % N
_CORR = _np.zeros((N, 1), _np.float32)
_CORR[_SPREAD, 0] = float(NT)


def kernel(x, edge_index, W1, b1, g1, be1, a1, W2, b2, g2, be2, a2):
    src = edge_index[0].astype(jnp.int32)
    dst = edge_index[1].astype(jnp.int32)

    trash = N + (jnp.arange(_NSLOT, dtype=jnp.int32) % (NPAD - N))
    spread = jnp.asarray(_SPREAD, dtype=jnp.int32)
    src_g = _tile_slabs(src, spread)
    dst_t = _tile_slabs(dst, trash)
    pair = jnp.stack([src_g, dst_t], axis=2)

    ones_cw = jnp.ones((BLK, CW), F32)
    zeros_cw = jnp.zeros((SLICE, CW), F32)
    corr = jnp.asarray(_CORR)

    degs = _deg_call(pair, ones_cw, zeros_cw)
    nb, bn = 25, 400

    hs = _prescale(x, degs, corr, 2, nb, bn)
    agg1 = _mp_call(1, hs, pair, zeros_cw)

    h1s = _dense(agg1, W1, b1.reshape(1, -1), g1.reshape(1, -1),
                 be1.reshape(1, -1), a1.reshape(1, 1), degs, corr,
                 out_chunks=4, nb=nb, bn=bn)

    agg2 = _mp_call(2, h1s, pair, zeros_cw)

    out = _dense(agg2, W2, b2.reshape(1, -1), g2.reshape(1, -1),
                 be2.reshape(1, -1), a2.reshape(1, 1), degs, corr,
                 out_chunks=0, nb=nb, bn=bn)
    return out


# speedup vs baseline: 1.0967x; 1.0004x over previous
import functools

import numpy as _np

import jax
import jax.numpy as jnp
from jax import lax
from jax.experimental import pallas as pl
from jax.experimental.pallas import tpu as pltpu
from jax.experimental.pallas import tpu_sc as plsc

N = 10000
E = 160000
NT = 16
PT = E // NT
BLK = 128
NBLK = 80
NPAD = 10032
RPT = NPAD // NT
SLICE = 640
CW = 128
F32 = jnp.float32

_K = 3
_M = 6


def _slice_start(sid):
    st = sid * RPT
    st = (st // 8) * 8
    return jnp.minimum(st, NPAD - SLICE)


def _sc_mesh():
    return plsc.VectorSubcoreMesh(core_axis_name="c", subcore_axis_name="s")


def _deg_body(slab, ones_h, zeros_h, out, idx_v, ones_v, acc, sem):
    core = lax.axis_index("c")
    sid = lax.axis_index("s")
    pltpu.sync_copy(slab.at[sid], idx_v)
    pltpu.sync_copy(ones_h, ones_v)
    st = _slice_start(sid)
    pltpu.sync_copy(zeros_h, acc.at[pl.ds(st, SLICE)])
    plsc.subcore_barrier()

    def start(j):
        pltpu.async_copy(ones_v, acc.at[idx_v.at[j, core]], sem, add=True)

    def drain(j):
        pltpu.make_async_copy(ones_v, acc.at[idx_v.at[j, core]], sem).wait()

    W = 8
    for j in range(W):
        start(j)

    def step(g, carry):
        j = W + g
        drain(j - W)
        start(j)
        return carry

    lax.fori_loop(0, NBLK - W, step, 0)
    for j in range(NBLK - W, NBLK):
        drain(j)
    plsc.subcore_barrier()
    pltpu.sync_copy(acc.at[pl.ds(st, SLICE)],
                    out.at[core, pl.ds(st, SLICE)])


def _deg_call(slab, ones_h, zeros_h):
    return pl.kernel(
        _deg_body,
        out_type=jax.ShapeDtypeStruct((2, NPAD, CW), F32),
        mesh=_sc_mesh(),
        scratch_types=[
            pltpu.VMEM((NBLK, 2, BLK), jnp.int32),
            pltpu.VMEM((BLK, CW), F32),
            pltpu.VMEM_SHARED((NPAD, CW), F32),
            pltpu.SemaphoreType.DMA,
        ],
    )(slab, ones_h, zeros_h)


def _mp_body(P, tab, pair_slab, zeros_h, out,
             r0, r1, r2, q0, q1, q2, q3, q4, q5, acc, gsems, ssems, psems):
    rows = [r0, r1, r2]
    pairs = [q0, q1, q2, q3, q4, q5]
    core = lax.axis_index("c")
    sid = lax.axis_index("s")

    base = pair_slab.at[sid]
    for p in range(P):
        chunk = 2 * p + core

        def p_start(j, m):
            pltpu.async_copy(base.at[j], pairs[m], psems.at[m])

        def p_wait(j, m):
            pltpu.make_async_copy(base.at[j], pairs[m], psems.at[m]).wait()

        def g_start(b, m):
            pltpu.async_copy(tab.at[chunk].at[pairs[m].at[0]], rows[b],
                             gsems.at[b])

        def g_wait(b, m):
            pltpu.make_async_copy(tab.at[chunk].at[pairs[m].at[0]], rows[b],
                                  gsems.at[b]).wait()

        def s_start(b, m):
            pltpu.async_copy(rows[b], acc.at[pairs[m].at[1]], ssems.at[b],
                             add=True)

        def s_wait(b, m):
            pltpu.make_async_copy(rows[b], acc.at[pairs[m].at[1]],
                                  ssems.at[b]).wait()

        st = _slice_start(sid)
        if p:
            plsc.subcore_barrier()
        pltpu.sync_copy(zeros_h, acc.at[pl.ds(st, SLICE)])
        plsc.subcore_barrier()

        for j in range(3):
            p_start(j, j)
        p_wait(0, 0); g_start(0, 0); p_start(3, 3)
        p_wait(1, 1); g_start(1, 1); p_start(4, 4)
        p_wait(2, 2); g_start(2, 2); p_start(5, 5)
        g_wait(0, 0); s_start(0, 0)

        def step6(g, carry):
            j0 = 3 + 6 * g
            for u in range(6):
                ju = j0 + u
                b, m = u % _K, (3 + u) % _M
                s_wait(u % _K, u % _M)
                p_start(ju + 3, u % _M)
                p_wait(ju, m)
                g_start(b, m)
                g_wait((u + 1) % _K, (1 + u) % _M)
                s_start((u + 1) % _K, (1 + u) % _M)
            return carry

        lax.fori_loop(0, 12, step6, 0)

        for j in range(75, NBLK):
            s_wait((j - 3) % _K, (j - 3) % _M)
            if j + 3 < NBLK:
                p_start(j + 3, (j + 3) % _M)
            p_wait(j, j % _M)
            g_start(j % _K, j % _M)
            g_wait((j - 2) % _K, (j - 2) % _M)
            s_start((j - 2) % _K, (j - 2) % _M)
        for j in range(NBLK - 2, NBLK):
            g_wait(j % _K, j % _M)
            s_start(j % _K, j % _M)
        for j in range(NBLK - 3, NBLK):
            s_wait(j % _K, j % _M)

        plsc.subcore_barrier()
        pltpu.sync_copy(acc.at[pl.ds(st, SLICE)],
                        out.at[chunk, pl.ds(st, SLICE)])


def _mp_call(P, tab, pair_slab, zeros_h):
    nch = 2 * P
    return pl.kernel(
        functools.partial(_mp_body, P),
        out_type=jax.ShapeDtypeStruct((nch, NPAD, CW), F32),
        mesh=_sc_mesh(),
        scratch_types=(
            [pltpu.VMEM((BLK, CW), F32)] * _K
            + [pltpu.VMEM((2, BLK), jnp.int32)] * _M
            + [
                pltpu.VMEM_SHARED((NPAD, CW), F32),
                pltpu.SemaphoreType.DMA((_K,)),
                pltpu.SemaphoreType.DMA((_K,)),
                pltpu.SemaphoreType.DMA((_M,)),
            ]
        ),
    )(tab, pair_slab, zeros_h)


def _prescale_body(x_ref, degs_ref, corr_ref, o_ref):
    d = degs_ref[0][:, 0:1] - corr_ref[...]
    s = lax.rsqrt(jnp.maximum(d, 1.0))
    o_ref[0] = x_ref[...] * s


def _prescale(x, degs, corr, n_chunks, nb, bn):
    return pl.pallas_call(
        _prescale_body,
        grid=(nb, n_chunks),
        in_specs=[
            pl.BlockSpec((bn, CW), lambda i, c: (i, c)),
            pl.BlockSpec((1, bn, CW), lambda i, c: (0, i, 0)),
            pl.BlockSpec((bn, 1), lambda i, c: (i, 0)),
        ],
        out_specs=pl.BlockSpec((1, bn, CW), lambda i, c: (c, i, 0)),
        out_shape=jax.ShapeDtypeStruct((n_chunks, N, CW), F32),
    )(x, degs, corr)


def _dense_body(nch_in, out_chunks, agg_ref, w_ref, b_ref, g_ref, be_ref,
                a_ref, degs_ref, corr_ref, o_ref):
    t = jnp.concatenate([agg_ref[c] for c in range(nch_in)], axis=1)
    acc = jnp.dot(t.astype(jnp.bfloat16), w_ref[...].astype(jnp.bfloat16),
                  preferred_element_type=F32)
    s_in = lax.rsqrt(jnp.maximum(degs_ref[1][:, 0:1], 1.0))
    acc = acc * s_in + b_ref[...]
    mu = jnp.mean(acc, axis=-1, keepdims=True)
    d = acc - mu
    var = jnp.mean(d * d, axis=-1, keepdims=True)
    u = d * lax.rsqrt(var + 1e-5) * g_ref[...] + be_ref[...]
    r = jnp.where(u >= 0, u, a_ref[0, 0] * u)
    if out_chunks:
        s_out = lax.rsqrt(jnp.maximum(degs_ref[0][:, 0:1] - corr_ref[...], 1.0))
        r = r * s_out
        for c2 in range(out_chunks):
            o_ref[c2] = r[:, c2 * CW:(c2 + 1) * CW]
    else:
        o_ref[...] = r


def _dense(agg, W, b, g, be, a, degs, corr, out_chunks, nb, bn):
    nch_in = agg.shape[0]
    dout = W.shape[1]
    if out_chunks:
        out_shape = jax.ShapeDtypeStruct((out_chunks, N, CW), F32)
        out_spec = pl.BlockSpec((out_chunks, bn, CW), lambda i: (0, i, 0))
    else:
        out_shape = jax.ShapeDtypeStruct((N, dout), F32)
        out_spec = pl.BlockSpec((bn, dout), lambda i: (i, 0))
    return pl.pallas_call(
        functools.partial(_dense_body, nch_in, out_chunks),
        grid=(nb,),
        in_specs=[
            pl.BlockSpec((nch_in, bn, CW), lambda i: (0, i, 0)),
            pl.BlockSpec(W.shape, lambda i: (0, 0)),
            pl.BlockSpec((1, dout), lambda i: (0, 0)),
            pl.BlockSpec((1, dout), lambda i: (0, 0)),
            pl.BlockSpec((1, dout), lambda i: (0, 0)),
            pl.BlockSpec((1, 1), lambda i: (0, 0)),
            pl.BlockSpec((2, bn, CW), lambda i: (0, i, 0)),
            pl.BlockSpec((bn, 1), lambda i: (i, 0)),
        ],
        out_specs=out_spec,
        out_shape=out_shape,
    )(agg, W, b, g, be, a, degs, corr)


def _tile_slabs(idx, pad_vals):
    tiles = idx.reshape(NT, PT)
    pad = jnp.broadcast_to(pad_vals[None, :], (NT, pad_vals.shape[0]))
    return jnp.concatenate([tiles, pad], axis=1).reshape(NT, NBLK, BLK)


_NSLOT = NBLK * BLK - PT
_SPREAD = (_np.arange(_NSLOT) * 37) % N
_CORR = _np.zeros((N, 1), _np.float32)
_CORR[_SPREAD, 0] = float(NT)


def kernel(x, edge_index, W1, b1, g1, be1, a1, W2, b2, g2, be2, a2):
    src = edge_index[0].astype(jnp.int32)
    dst = edge_index[1].astype(jnp.int32)

    trash = N + (jnp.arange(_NSLOT, dtype=jnp.int32) % (NPAD - N))
    spread = jnp.asarray(_SPREAD, dtype=jnp.int32)
    src_g = _tile_slabs(src, spread)
    dst_t = _tile_slabs(dst, trash)
    pair = jnp.stack([src_g, dst_t], axis=2)

    ones_cw = jnp.ones((BLK, CW), F32)
    zeros_cw = jnp.zeros((SLICE, CW), F32)
    corr = jnp.asarray(_CORR)

    degs = _deg_call(pair, ones_cw, zeros_cw)
    nb, bn = 25, 400

    hs = _prescale(x, degs, corr, 2, nb, bn)
    agg1 = _mp_call(1, hs, pair, zeros_cw)

    h1s = _dense(agg1, W1, b1.reshape(1, -1), g1.reshape(1, -1),
                 be1.reshape(1, -1), a1.reshape(1, 1), degs, corr,
                 out_chunks=4, nb=nb, bn=bn)

    agg2 = _mp_call(2, h1s, pair, zeros_cw)

    out = _dense(agg2, W2, b2.reshape(1, -1), g2.reshape(1, -1),
                 be2.reshape(1, -1), a2.reshape(1, 1), degs, corr,
                 out_chunks=0, nb=nb, bn=bn)
    return out


# speedup vs baseline: 1.0969x; 1.0002x over previous
import functools

import numpy as _np

import jax
import jax.numpy as jnp
from jax import lax
from jax.experimental import pallas as pl
from jax.experimental.pallas import tpu as pltpu
from jax.experimental.pallas import tpu_sc as plsc

N = 10000
E = 160000
NT = 16
PT = E // NT
BLK = 128
NBLK = 80
NPAD = 10032
RPT = NPAD // NT
SLICE = 640
CW = 128
F32 = jnp.float32

_K = 3
_M = 6


def _slice_start(sid):
    st = sid * RPT
    st = (st // 8) * 8
    return jnp.minimum(st, NPAD - SLICE)


def _sc_mesh():
    return plsc.VectorSubcoreMesh(core_axis_name="c", subcore_axis_name="s")


def _deg_body(slab, ones_h, zeros_h, out, idx_v, ones_v, acc, sem):
    core = lax.axis_index("c")
    sid = lax.axis_index("s")
    pltpu.sync_copy(slab.at[sid], idx_v)
    pltpu.sync_copy(ones_h, ones_v)
    st = _slice_start(sid)
    pltpu.sync_copy(zeros_h, acc.at[pl.ds(st, SLICE)])
    plsc.subcore_barrier()

    def start(j):
        pltpu.async_copy(ones_v, acc.at[idx_v.at[j, core]], sem, add=True)

    def drain(j):
        pltpu.make_async_copy(ones_v, acc.at[idx_v.at[j, core]], sem).wait()

    W = 8
    for j in range(W):
        start(j)

    def step(g, carry):
        j = W + g
        drain(j - W)
        start(j)
        return carry

    lax.fori_loop(0, NBLK - W, step, 0)
    for j in range(NBLK - W, NBLK):
        drain(j)
    plsc.subcore_barrier()
    pltpu.sync_copy(acc.at[pl.ds(st, SLICE)],
                    out.at[core, pl.ds(st, SLICE)])


def _deg_call(slab, ones_h, zeros_h):
    return pl.kernel(
        _deg_body,
        out_type=jax.ShapeDtypeStruct((2, NPAD, CW), F32),
        mesh=_sc_mesh(),
        scratch_types=[
            pltpu.VMEM((NBLK, 2, BLK), jnp.int32),
            pltpu.VMEM((BLK, CW), F32),
            pltpu.VMEM_SHARED((NPAD, CW), F32),
            pltpu.SemaphoreType.DMA,
        ],
    )(slab, ones_h, zeros_h)


def _mp_body(P, tab, pair_slab, zeros_h, out,
             r0, r1, r2, q0, q1, q2, q3, q4, q5, acc, gsems, ssems, psems):
    rows = [r0, r1, r2]
    pairs = [q0, q1, q2, q3, q4, q5]
    core = lax.axis_index("c")
    sid = lax.axis_index("s")

    base = pair_slab.at[sid]
    for p in range(P):
        chunk = 2 * p + core

        def p_start(j, m):
            pltpu.async_copy(base.at[j], pairs[m], psems.at[m])

        def p_wait(j, m):
            pltpu.make_async_copy(base.at[j], pairs[m], psems.at[m]).wait()

        def g_start(b, m):
            pltpu.async_copy(tab.at[chunk].at[pairs[m].at[0]], rows[b],
                             gsems.at[b])

        def g_wait(b, m):
            pltpu.make_async_copy(tab.at[chunk].at[pairs[m].at[0]], rows[b],
                                  gsems.at[b]).wait()

        def s_start(b, m):
            pltpu.async_copy(rows[b], acc.at[pairs[m].at[1]], ssems.at[b],
                             add=True)

        def s_wait(b, m):
            pltpu.make_async_copy(rows[b], acc.at[pairs[m].at[1]],
                                  ssems.at[b]).wait()

        st = _slice_start(sid)
        if p:
            plsc.subcore_barrier()
        pltpu.sync_copy(zeros_h, acc.at[pl.ds(st, SLICE)])
        plsc.subcore_barrier()

        for j in range(3):
            p_start(j, j)
        p_wait(0, 0); g_start(0, 0); p_start(3, 3)
        p_wait(1, 1); g_start(1, 1); p_start(4, 4)
        p_wait(2, 2); g_start(2, 2); p_start(5, 5)
        g_wait(0, 0); s_start(0, 0)

        def step6(g, carry):
            j0 = 3 + 6 * g
            for u in range(6):
                ju = j0 + u
                b, m = u % _K, (3 + u) % _M
                s_wait(u % _K, u % _M)
                p_start(ju + 3, u % _M)
                p_wait(ju, m)
                g_start(b, m)
                g_wait((u + 1) % _K, (1 + u) % _M)
                s_start((u + 1) % _K, (1 + u) % _M)
            return carry

        lax.fori_loop(0, 12, step6, 0)

        for j in range(75, NBLK):
            s_wait((j - 3) % _K, (j - 3) % _M)
            if j + 3 < NBLK:
                p_start(j + 3, (j + 3) % _M)
            p_wait(j, j % _M)
            g_start(j % _K, j % _M)
            g_wait((j - 2) % _K, (j - 2) % _M)
            s_start((j - 2) % _K, (j - 2) % _M)
        for j in range(NBLK - 2, NBLK):
            g_wait(j % _K, j % _M)
            s_start(j % _K, j % _M)
        for j in range(NBLK - 3, NBLK):
            s_wait(j % _K, j % _M)

        plsc.subcore_barrier()
        pltpu.sync_copy(acc.at[pl.ds(st, SLICE)],
                        out.at[chunk, pl.ds(st, SLICE)])


def _mp_call(P, tab, pair_slab, zeros_h):
    nch = 2 * P
    return pl.kernel(
        functools.partial(_mp_body, P),
        out_type=jax.ShapeDtypeStruct((nch, NPAD, CW), F32),
        mesh=_sc_mesh(),
        scratch_types=(
            [pltpu.VMEM((BLK, CW), F32)] * _K
            + [pltpu.VMEM((2, BLK), jnp.int32)] * _M
            + [
                pltpu.VMEM_SHARED((NPAD, CW), F32),
                pltpu.SemaphoreType.DMA((_K,)),
                pltpu.SemaphoreType.DMA((_K,)),
                pltpu.SemaphoreType.DMA((_M,)),
            ]
        ),
    )(tab, pair_slab, zeros_h)


def _prescale_body(x_ref, degs_ref, corr_ref, o_ref):
    d = degs_ref[0][:, 0:1] - corr_ref[...]
    s = lax.rsqrt(jnp.maximum(d, 1.0))
    o_ref[0] = x_ref[...] * s


def _prescale(x, degs, corr, n_chunks, nb, bn):
    return pl.pallas_call(
        _prescale_body,
        grid=(nb, n_chunks),
        in_specs=[
            pl.BlockSpec((bn, CW), lambda i, c: (i, c)),
            pl.BlockSpec((1, bn, CW), lambda i, c: (0, i, 0)),
            pl.BlockSpec((bn, 1), lambda i, c: (i, 0)),
        ],
        out_specs=pl.BlockSpec((1, bn, CW), lambda i, c: (c, i, 0)),
        out_shape=jax.ShapeDtypeStruct((n_chunks, N, CW), F32),
    )(x, degs, corr)


def _dense_body(nch_in, out_chunks, agg_ref, w_ref, b_ref, g_ref, be_ref,
                a_ref, degs_ref, corr_ref, o_ref):
    t = jnp.concatenate([agg_ref[c] for c in range(nch_in)], axis=1)
    acc = jnp.dot(t, w_ref[...], preferred_element_type=F32)
    s_in = lax.rsqrt(jnp.maximum(degs_ref[1][:, 0:1], 1.0))
    acc = acc * s_in + b_ref[...]
    mu = jnp.mean(acc, axis=-1, keepdims=True)
    d = acc - mu
    var = jnp.mean(d * d, axis=-1, keepdims=True)
    u = d * lax.rsqrt(var + 1e-5) * g_ref[...] + be_ref[...]
    r = jnp.where(u >= 0, u, a_ref[0, 0] * u)
    if out_chunks:
        s_out = lax.rsqrt(jnp.maximum(degs_ref[0][:, 0:1] - corr_ref[...], 1.0))
        r = r * s_out
        for c2 in range(out_chunks):
            o_ref[c2] = r[:, c2 * CW:(c2 + 1) * CW]
    else:
        o_ref[...] = r


def _dense(agg, W, b, g, be, a, degs, corr, out_chunks, nb, bn):
    nch_in = agg.shape[0]
    dout = W.shape[1]
    if out_chunks:
        out_shape = jax.ShapeDtypeStruct((out_chunks, N, CW), F32)
        out_spec = pl.BlockSpec((out_chunks, bn, CW), lambda i: (0, i, 0))
    else:
        out_shape = jax.ShapeDtypeStruct((N, dout), F32)
        out_spec = pl.BlockSpec((bn, dout), lambda i: (i, 0))
    return pl.pallas_call(
        functools.partial(_dense_body, nch_in, out_chunks),
        grid=(nb,),
        in_specs=[
            pl.BlockSpec((nch_in, bn, CW), lambda i: (0, i, 0)),
            pl.BlockSpec(W.shape, lambda i: (0, 0)),
            pl.BlockSpec((1, dout), lambda i: (0, 0)),
            pl.BlockSpec((1, dout), lambda i: (0, 0)),
            pl.BlockSpec((1, dout), lambda i: (0, 0)),
            pl.BlockSpec((1, 1), lambda i: (0, 0)),
            pl.BlockSpec((2, bn, CW), lambda i: (0, i, 0)),
            pl.BlockSpec((bn, 1), lambda i: (i, 0)),
        ],
        out_specs=out_spec,
        out_shape=out_shape,
    )(agg, W, b, g, be, a, degs, corr)


def _tile_slabs(idx, pad_vals):
    tiles = idx.reshape(NT, PT)
    pad = jnp.broadcast_to(pad_vals[None, :], (NT, pad_vals.shape[0]))
    return jnp.concatenate([tiles, pad], axis=1).reshape(NT, NBLK, BLK)


_NSLOT = NBLK * BLK - PT
_SPREAD = (_np.arange(_NSLOT) * 37) % N
_CORR = _np.zeros((N, 1), _np.float32)
_CORR[_SPREAD, 0] = float(NT)


def kernel(x, edge_index, W1, b1, g1, be1, a1, W2, b2, g2, be2, a2):
    src = edge_index[0].astype(jnp.int32)
    dst = edge_index[1].astype(jnp.int32)

    trash = N + (jnp.arange(_NSLOT, dtype=jnp.int32) % (NPAD - N))
    spread = jnp.asarray(_SPREAD, dtype=jnp.int32)
    src_g = _tile_slabs(src, spread)
    dst_t = _tile_slabs(dst, trash)
    pair = jnp.stack([src_g, dst_t], axis=2)

    ones_cw = jnp.ones((BLK, CW), F32)
    zeros_cw = jnp.zeros((SLICE, CW), F32)
    corr = jnp.asarray(_CORR)

    degs = _deg_call(pair, ones_cw, zeros_cw)
    nb, bn = 25, 400

    hs = _prescale(x, degs, corr, 2, nb, bn)
    agg1 = _mp_call(1, hs, pair, zeros_cw)

    h1s = _dense(agg1, W1, b1.reshape(1, -1), g1.reshape(1, -1),
                 be1.reshape(1, -1), a1.reshape(1, 1), degs, corr,
                 out_chunks=4, nb=nb, bn=bn)

    agg2 = _mp_call(2, h1s, pair, zeros_cw)

    out = _dense(agg2, W2, b2.reshape(1, -1), g2.reshape(1, -1),
                 be2.reshape(1, -1), a2.reshape(1, 1), degs, corr,
                 out_chunks=0, nb=nb, bn=bn)
    return out
